# baseline (speedup 1.0000x reference)
import jax
import jax.numpy as jnp
from jax.experimental import pallas as pl
from jax.experimental.pallas import tpu as pltpu

N_LAYER = 24
N_HEAD = 16
HEAD_SIZE = 64
TOTAL_DIM = 3072
OUT_DIM = TOTAL_DIM // 3
T = 4096
T_CHUNKS = 8


def _fused_body(e_ref, w_ref, b_ref, out_ref, vec_ref):
    t = pl.program_id(0)

    @pl.when(t == 0)
    def _init():
        vec_ref[...] = jnp.zeros_like(vec_ref)

    @pl.when(t < T_CHUNKS)
    def _mean():
        ones = jnp.full((1, T // T_CHUNKS), 1.0 / T, dtype=jnp.float32)
        vec_ref[...] += jax.lax.dot_general(
            ones, e_ref[...], (((1,), (0,)), ((), ())),
            preferred_element_type=jnp.float32,
        )

    @pl.when(t >= T_CHUNKS)
    def _matvec():
        out_ref[0] = (
            jax.lax.dot_general(
                vec_ref[...], w_ref[0], (((1,), (1,)), ((), ())),
                preferred_element_type=jnp.float32,
            )
            + b_ref[0]
        )


def kernel(system_emb, W_proj, b_proj):
    e = system_emb.reshape(T, TOTAL_DIM)
    out = pl.pallas_call(
        _fused_body,
        grid=(T_CHUNKS + N_LAYER,),
        in_specs=[
            pl.BlockSpec(
                (T // T_CHUNKS, TOTAL_DIM),
                lambda t: (jnp.minimum(t, T_CHUNKS - 1), 0),
            ),
            pl.BlockSpec(
                (1, OUT_DIM, TOTAL_DIM),
                lambda t: (jnp.maximum(t - T_CHUNKS, 0), 0, 0),
            ),
            pl.BlockSpec(
                (1, 1, OUT_DIM),
                lambda t: (jnp.maximum(t - T_CHUNKS, 0), 0, 0),
            ),
        ],
        out_specs=pl.BlockSpec(
            (1, 1, OUT_DIM),
            lambda t: (jnp.maximum(t - T_CHUNKS, 0), 0, 0),
        ),
        out_shape=jax.ShapeDtypeStruct((N_LAYER, 1, OUT_DIM), jnp.float32),
        scratch_shapes=[pltpu.VMEM((1, TOTAL_DIM), jnp.float32)],
    )(e, W_proj, b_proj.reshape(N_LAYER, 1, OUT_DIM))
    return out.reshape(N_LAYER, N_HEAD, HEAD_SIZE)


# speedup vs baseline: 1.0036x; 1.0036x over previous
import jax
import jax.numpy as jnp
from jax.experimental import pallas as pl
from jax.experimental.pallas import tpu as pltpu

N_LAYER = 24
N_HEAD = 16
HEAD_SIZE = 64
TOTAL_DIM = 3072
OUT_DIM = TOTAL_DIM // 3
T = 4096
T_CHUNKS = 8


def _fused_body(e_ref, w_ref, b_ref, out_ref, vec_ref):
    t = pl.program_id(0)

    @pl.when(t == 0)
    def _init():
        vec_ref[...] = jnp.zeros_like(vec_ref)

    @pl.when(t < T_CHUNKS)
    def _mean():
        vec_ref[...] += jnp.sum(e_ref[...], axis=0, keepdims=True) * (1.0 / T)

    @pl.when(t >= T_CHUNKS)
    def _matvec():
        out_ref[0] = (
            jax.lax.dot_general(
                vec_ref[...], w_ref[0], (((1,), (1,)), ((), ())),
                preferred_element_type=jnp.float32,
            )
            + b_ref[0]
        )


def kernel(system_emb, W_proj, b_proj):
    e = system_emb.reshape(T, TOTAL_DIM)
    out = pl.pallas_call(
        _fused_body,
        grid=(T_CHUNKS + N_LAYER,),
        in_specs=[
            pl.BlockSpec(
                (T // T_CHUNKS, TOTAL_DIM),
                lambda t: (jnp.minimum(t, T_CHUNKS - 1), 0),
            ),
            pl.BlockSpec(
                (1, OUT_DIM, TOTAL_DIM),
                lambda t: (jnp.maximum(t - T_CHUNKS, 0), 0, 0),
            ),
            pl.BlockSpec(
                (1, 1, OUT_DIM),
                lambda t: (jnp.maximum(t - T_CHUNKS, 0), 0, 0),
            ),
        ],
        out_specs=pl.BlockSpec(
            (1, 1, OUT_DIM),
            lambda t: (jnp.maximum(t - T_CHUNKS, 0), 0, 0),
        ),
        out_shape=jax.ShapeDtypeStruct((N_LAYER, 1, OUT_DIM), jnp.float32),
        scratch_shapes=[pltpu.VMEM((1, TOTAL_DIM), jnp.float32)],
    )(e, W_proj, b_proj.reshape(N_LAYER, 1, OUT_DIM))
    return out.reshape(N_LAYER, N_HEAD, HEAD_SIZE)
